# Initial kernel scaffold; baseline (speedup 1.0000x reference)
import jax
import jax.numpy as jnp
from jax.experimental import pallas as pl


def kernel(new_vertices, vertices, points_feat):
    raise NotImplementedError("write your pallas kernel here")


# speedup vs baseline: 8.1567x; 8.1567x over previous
import jax
import jax.numpy as jnp
from jax.experimental import pallas as pl

_TQ = 256


def _topk_interp_body(r1_ref, r2_ref, qp_ref, vtp_ref, feat_ref, out_ref):
    dot = jax.lax.dot_general(
        qp_ref[...], vtp_ref[...], (((1,), (0,)), ((), ())),
        preferred_element_type=jnp.float32)
    d = (r1_ref[...] + r2_ref[...]) - 2.0 * dot
    kdim = d.shape[1]
    iota = jax.lax.broadcasted_iota(jnp.int32, d.shape, 1)
    mins, idxs = [], []
    dd = d
    for _ in range(3):
        m = jnp.min(dd, axis=1, keepdims=True)
        first = jnp.min(jnp.where(dd == m, iota, kdim), axis=1, keepdims=True)
        mins.append(m)
        idxs.append(first)
        dd = jnp.where(iota == first, jnp.float32(jnp.inf), dd)
    d1, d2, d3 = mins
    e1 = jnp.ones_like(d1)
    e2 = jnp.exp(d1 - d2)
    e3 = jnp.exp(d1 - d3)
    s = e1 + e2 + e3
    w = (jnp.where(iota == idxs[0], e1 / s, 0.0)
         + jnp.where(iota == idxs[1], e2 / s, 0.0)
         + jnp.where(iota == idxs[2], e3 / s, 0.0))
    out_ref[...] = jax.lax.dot_general(
        w, feat_ref[...], (((1,), (0,)), ((), ())),
        preferred_element_type=jnp.float32,
        precision=jax.lax.Precision.HIGHEST)


def kernel(new_vertices, vertices, points_feat):
    q_total = new_vertices.shape[0]
    k_total, c_dim = points_feat.shape[1], points_feat.shape[2]
    r1 = jnp.sum(new_vertices ** 2, axis=-1)[:, None]
    r2 = jnp.sum(vertices ** 2, axis=-1)[None, :]
    qp = jnp.pad(new_vertices, ((0, 0), (0, 5))).astype(jnp.bfloat16)
    vtp = jnp.pad(vertices.T, ((0, 5), (0, 0))).astype(jnp.bfloat16)
    feat = points_feat[0]
    out = pl.pallas_call(
        _topk_interp_body,
        grid=(q_total // _TQ,),
        in_specs=[
            pl.BlockSpec((_TQ, 1), lambda i: (i, 0)),
            pl.BlockSpec((1, k_total), lambda i: (0, 0)),
            pl.BlockSpec((_TQ, 8), lambda i: (i, 0)),
            pl.BlockSpec((8, k_total), lambda i: (0, 0)),
            pl.BlockSpec((k_total, c_dim), lambda i: (0, 0)),
        ],
        out_specs=pl.BlockSpec((_TQ, c_dim), lambda i: (i, 0)),
        out_shape=jax.ShapeDtypeStruct((q_total, c_dim), jnp.float32),
    )(r1, r2, qp, vtp, feat)
    return out[None]


# speedup vs baseline: 13.0234x; 1.5967x over previous
import jax
import jax.numpy as jnp
from jax.experimental import pallas as pl

_TQ = 256


def _topk_interp_body(r1_ref, r2_ref, qp_ref, vtp_ref, feat_ref, out_ref):
    dot = jax.lax.dot_general(
        qp_ref[...], vtp_ref[...], (((1,), (0,)), ((), ())),
        preferred_element_type=jnp.float32)
    d = (r1_ref[...] + r2_ref[...]) - 2.0 * dot
    kdim = d.shape[1]
    iota = jax.lax.broadcasted_iota(jnp.int32, d.shape, 1)
    mins, idxs = [], []
    dd = d
    for _ in range(3):
        m = jnp.min(dd, axis=1, keepdims=True)
        first = jnp.min(jnp.where(dd == m, iota, kdim), axis=1, keepdims=True)
        mins.append(m)
        idxs.append(first)
        dd = jnp.where(iota == first, jnp.float32(jnp.inf), dd)
    d1, d2, d3 = mins
    e1 = jnp.ones_like(d1)
    e2 = jnp.exp(d1 - d2)
    e3 = jnp.exp(d1 - d3)
    s = e1 + e2 + e3
    w = (jnp.where(iota == idxs[0], e1 / s, 0.0)
         + jnp.where(iota == idxs[1], e2 / s, 0.0)
         + jnp.where(iota == idxs[2], e3 / s, 0.0))
    out_ref[...] = jax.lax.dot_general(
        w, feat_ref[...], (((1,), (0,)), ((), ())),
        preferred_element_type=jnp.float32)


def kernel(new_vertices, vertices, points_feat):
    q_total = new_vertices.shape[0]
    k_total, c_dim = points_feat.shape[1], points_feat.shape[2]
    r1 = jnp.sum(new_vertices ** 2, axis=-1)[:, None]
    r2 = jnp.sum(vertices ** 2, axis=-1)[None, :]
    qp = jnp.pad(new_vertices, ((0, 0), (0, 5))).astype(jnp.bfloat16)
    vtp = jnp.pad(vertices.T, ((0, 5), (0, 0))).astype(jnp.bfloat16)
    feat = points_feat[0]
    out = pl.pallas_call(
        _topk_interp_body,
        grid=(q_total // _TQ,),
        in_specs=[
            pl.BlockSpec((_TQ, 1), lambda i: (i, 0)),
            pl.BlockSpec((1, k_total), lambda i: (0, 0)),
            pl.BlockSpec((_TQ, 8), lambda i: (i, 0)),
            pl.BlockSpec((8, k_total), lambda i: (0, 0)),
            pl.BlockSpec((k_total, c_dim), lambda i: (0, 0)),
        ],
        out_specs=pl.BlockSpec((_TQ, c_dim), lambda i: (i, 0)),
        out_shape=jax.ShapeDtypeStruct((q_total, c_dim), jnp.float32),
    )(r1, r2, qp, vtp, feat)
    return out[None]


# speedup vs baseline: 14.9108x; 1.1449x over previous
import jax
import jax.numpy as jnp
from jax.experimental import pallas as pl

_TQ = 256


def _topk_interp_body(r1_ref, r2_ref, qp_ref, vtp_ref, feat_ref, out_ref):
    dot = jax.lax.dot_general(
        qp_ref[...], vtp_ref[...], (((1,), (0,)), ((), ())),
        preferred_element_type=jnp.float32)
    d = (r1_ref[...] + r2_ref[...]) - 2.0 * dot
    kdim = d.shape[1]
    fiota = jax.lax.broadcasted_iota(jnp.int32, d.shape, 1).astype(jnp.float32)
    fbig = jnp.float32(kdim)
    mins, idxs = [], []
    dd = d
    for r in range(3):
        m = jnp.min(dd, axis=1, keepdims=True)
        first = jnp.min(jnp.where(dd == m, fiota, fbig), axis=1, keepdims=True)
        mins.append(m)
        idxs.append(first)
        if r < 2:
            dd = jnp.where(fiota == first, jnp.float32(jnp.inf), dd)
    d1, d2, d3 = mins
    e1 = jnp.ones_like(d1)
    e2 = jnp.exp(d1 - d2)
    e3 = jnp.exp(d1 - d3)
    s = e1 + e2 + e3
    w = (jnp.where(fiota == idxs[0], e1 / s, 0.0)
         + jnp.where(fiota == idxs[1], e2 / s, 0.0)
         + jnp.where(fiota == idxs[2], e3 / s, 0.0))
    out_ref[...] = jax.lax.dot_general(
        w, feat_ref[...], (((1,), (0,)), ((), ())),
        preferred_element_type=jnp.float32)


def kernel(new_vertices, vertices, points_feat):
    q_total = new_vertices.shape[0]
    k_total, c_dim = points_feat.shape[1], points_feat.shape[2]
    r1 = jnp.sum(new_vertices ** 2, axis=-1)[:, None]
    r2 = jnp.sum(vertices ** 2, axis=-1)[None, :]
    qp = jnp.pad(new_vertices, ((0, 0), (0, 5))).astype(jnp.bfloat16)
    vtp = jnp.pad(vertices.T, ((0, 5), (0, 0))).astype(jnp.bfloat16)
    feat = points_feat[0]
    out = pl.pallas_call(
        _topk_interp_body,
        grid=(q_total // _TQ,),
        in_specs=[
            pl.BlockSpec((_TQ, 1), lambda i: (i, 0)),
            pl.BlockSpec((1, k_total), lambda i: (0, 0)),
            pl.BlockSpec((_TQ, 8), lambda i: (i, 0)),
            pl.BlockSpec((8, k_total), lambda i: (0, 0)),
            pl.BlockSpec((k_total, c_dim), lambda i: (0, 0)),
        ],
        out_specs=pl.BlockSpec((_TQ, c_dim), lambda i: (i, 0)),
        out_shape=jax.ShapeDtypeStruct((q_total, c_dim), jnp.float32),
    )(r1, r2, qp, vtp, feat)
    return out[None]
